# baseline (speedup 1.0000x reference)
import functools

import jax
import jax.numpy as jnp
import numpy as np
from jax import lax
from jax.experimental import pallas as pl
from jax.experimental.pallas import tpu as pltpu
from jax.experimental.pallas import tpu_sc as plsc

N = 10000
E = 320000
DIN = 128
DHID = 128
DOUT = 128
EDIM = 16
NCLS = 10

NC = 2
NS = 16
NW = NC * NS
CH = 80
CPT = 128
GRP = 16
E_PAD = NW * CPT * CH
NCHUNKS = NW * CPT
N_ACC = 10112
RPT = N_ACC // NS

_f32 = jnp.float32


def _nfw_tc(x_ref, w_ref, o_ref):
    o_ref[...] = jnp.dot(x_ref[...], w_ref[...], preferred_element_type=_f32)


def _efw_tc(ef_ref, w1_ref, b1_ref, w2_ref, b2_ref, o1_ref, o2_ref):
    ef = ef_ref[...]
    o1_ref[...] = jnp.dot(ef, w1_ref[...], preferred_element_type=_f32) + b1_ref[...]
    o2_ref[...] = jnp.dot(ef, w2_ref[...], preferred_element_type=_f32) + b2_ref[...]


def _mid_tc(nf_ref, sp_ref, cp_ref, waxT_ref, wahT_ref,
            ba_ref, wm2xT_ref, h1_o, nfw2_o, inv_o):
    cnt = cp_ref[0][:, 0:1] + cp_ref[1][:, 0:1]
    inv = 1.0 / jnp.maximum(cnt, 1.0)
    hn = (sp_ref[0] + sp_ref[1]) * inv
    h1 = jnp.maximum(
        jnp.dot(nf_ref[...], waxT_ref[...], preferred_element_type=_f32)
        + jnp.dot(hn, wahT_ref[...], preferred_element_type=_f32)
        + ba_ref[...], 0.0)
    h1_o[...] = h1
    nfw2_o[...] = jnp.dot(h1, wm2xT_ref[...], preferred_element_type=_f32)
    inv_o[...] = jnp.broadcast_to(inv, inv_o.shape)


def _fin_tc(h1_ref, sp_ref, inv_ref, waxT_ref, wahT_ref, ba_ref,
            wpuT_ref, wpvT_ref, bp_ref, pu_o, pv_o):
    hn = (sp_ref[0] + sp_ref[1]) * inv_ref[...][:, 0:1]
    h2 = jnp.maximum(
        jnp.dot(h1_ref[...], waxT_ref[...], preferred_element_type=_f32)
        + jnp.dot(hn, wahT_ref[...], preferred_element_type=_f32)
        + ba_ref[...], 0.0)
    pu_o[...] = jnp.dot(h2, wpuT_ref[...], preferred_element_type=_f32) + bp_ref[...]
    pv_o[...] = jnp.dot(h2, wpvT_ref[...], preferred_element_type=_f32)


_NBLK = 1264
_EBLK = 2000


def _row_spec(blk, cols):
    return pl.BlockSpec((blk, cols), lambda i: (i, 0))


def _pair_spec(blk, cols):
    return pl.BlockSpec((2, blk, cols), lambda i: (0, i, 0))


def _full_spec(r, c):
    return pl.BlockSpec((r, c), lambda i: (0, 0))


_MESH = plsc.VectorSubcoreMesh(core_axis_name="c", subcore_axis_name="s",
                               num_cores=NC, num_subcores=NS)


def _sage_sc_body(nfw, efw, srcs, dsts, z128,
                  out,
                  src_v, dst_v, g_a, e_a, g_b, e_b, acc,
                  sem_a, sem_b):
    c = lax.axis_index("c")
    s = lax.axis_index("s")
    wid = s * NC + c
    r0 = s * RPT

    pltpu.sync_copy(z128.at[pl.ds(r0, RPT)], acc.at[pl.ds(r0, RPT)])
    plsc.subcore_barrier()

    def _issue(jj, cbase, g_v, e_v, sem):
        base = (cbase + jj) * CH
        pltpu.async_copy(efw.at[pl.ds(base, CH)], e_v, sem)
        pltpu.async_copy(nfw.at[src_v.at[jj]], g_v, sem)

    def _wait(jj, cbase, g_v, e_v, sem):
        base = (cbase + jj) * CH
        pltpu.make_async_copy(efw.at[pl.ds(base, CH)], e_v, sem).wait()
        pltpu.make_async_copy(nfw.at[src_v.at[jj]], g_v, sem).wait()

    def _compute(g_v, e_v):
        def _row(i, carry2):
            for r in range(2):
                for k in range(8):
                    sl = pl.ds(k * 16, 16)
                    g_v[2 * i + r, sl] = jnp.maximum(
                        g_v[2 * i + r, sl] + e_v[2 * i + r, sl], 0.0)
            return carry2
        lax.fori_loop(0, CH // 2, _row, 0)

    def _group(g, carry):
        cbase = wid * CPT + g * GRP
        pltpu.sync_copy(srcs.at[pl.ds(cbase, GRP)], src_v)
        pltpu.sync_copy(dsts.at[pl.ds(cbase, GRP)], dst_v)
        _issue(0, cbase, g_a, e_a, sem_a)

        def _pair(t, carry1):
            ja = 2 * t
            jb = 2 * t + 1
            _issue(jb, cbase, g_b, e_b, sem_b)
            _wait(ja, cbase, g_a, e_a, sem_a)
            _compute(g_a, e_a)
            pltpu.sync_copy(g_a, acc.at[dst_v.at[ja]], add=True)

            @pl.when(jb + 1 < GRP)
            def _():
                _issue(jb + 1, cbase, g_a, e_a, sem_a)

            _wait(jb, cbase, g_b, e_b, sem_b)
            _compute(g_b, e_b)
            pltpu.sync_copy(g_b, acc.at[dst_v.at[jb]], add=True)
            return carry1
        lax.fori_loop(0, GRP // 2, _pair, 0)
        return carry
    lax.fori_loop(0, CPT // GRP, _group, 0)

    plsc.subcore_barrier()
    pltpu.sync_copy(acc.at[pl.ds(r0, RPT)], out.at[c, pl.ds(r0, RPT)])


_sage_sc = pl.kernel(
    _sage_sc_body,
    out_type=jax.ShapeDtypeStruct((NC, N_ACC, 128), _f32),
    mesh=_MESH,
    compiler_params=pltpu.CompilerParams(use_tc_tiling_on_sc=False),
    scratch_types=[
        pltpu.VMEM((GRP, CH), jnp.int32),
        pltpu.VMEM((GRP, CH), jnp.int32),
        pltpu.VMEM((CH, 128), _f32),
        pltpu.VMEM((CH, 128), _f32),
        pltpu.VMEM((CH, 128), _f32),
        pltpu.VMEM((CH, 128), _f32),
        pltpu.VMEM_SHARED((N_ACC, 128), _f32),
        pltpu.SemaphoreType.DMA,
        pltpu.SemaphoreType.DMA,
    ],
)


def _cnt_sc_body(dsts, z16, cnt_out, dst_v, ones_v, cacc):
    c = lax.axis_index("c")
    s = lax.axis_index("s")
    wid = s * NC + c
    r0 = s * RPT

    pltpu.sync_copy(z16.at[pl.ds(r0, RPT)], cacc.at[pl.ds(r0, RPT)])

    def _ones_row(i, carry):
        ones_v[i, :] = jnp.ones((16,), _f32)
        return carry
    lax.fori_loop(0, CH, _ones_row, 0)

    pltpu.sync_copy(dsts.at[pl.ds(wid * CPT, CPT)], dst_v)
    plsc.subcore_barrier()

    def _chunk(j, carry):
        pltpu.sync_copy(ones_v, cacc.at[dst_v.at[j]], add=True)
        return carry
    lax.fori_loop(0, CPT, _chunk, 0)

    plsc.subcore_barrier()
    pltpu.sync_copy(cacc.at[pl.ds(r0, RPT)], cnt_out.at[c, pl.ds(r0, RPT)])


_cnt_sc = pl.kernel(
    _cnt_sc_body,
    out_type=jax.ShapeDtypeStruct((NC, N_ACC, 16), _f32),
    mesh=_MESH,
    compiler_params=pltpu.CompilerParams(use_tc_tiling_on_sc=False),
    scratch_types=[
        pltpu.VMEM((CPT, CH), jnp.int32),
        pltpu.VMEM((CH, 16), _f32),
        pltpu.VMEM_SHARED((N_ACC, 16), _f32),
    ],
)


CHP = 128
CPTP = E_PAD // (NW * CHP)


def _pred_sc_body(pu, pv, srcs, dsts, out,
                  src_v, dst_v, a_a, b_a, a_b, b_b,
                  sem_a, sem_b, sem_oa, sem_ob):
    c = lax.axis_index("c")
    s = lax.axis_index("s")
    wid = s * NC + c

    pltpu.sync_copy(srcs.at[pl.ds(wid * CPTP, CPTP)], src_v)
    pltpu.sync_copy(dsts.at[pl.ds(wid * CPTP, CPTP)], dst_v)

    def _issue(jj, a_v, b_v, sem):
        pltpu.async_copy(pu.at[src_v.at[jj]], a_v, sem)
        pltpu.async_copy(pv.at[dst_v.at[jj]], b_v, sem)

    def _wait(jj, a_v, b_v, sem):
        pltpu.make_async_copy(pu.at[src_v.at[jj]], a_v, sem).wait()
        pltpu.make_async_copy(pv.at[dst_v.at[jj]], b_v, sem).wait()

    def _add(a_v, b_v):
        def _row(i, carry2):
            for r in range(2):
                a_v[2 * i + r, :] = a_v[2 * i + r, :] + b_v[2 * i + r, :]
            return carry2
        lax.fori_loop(0, CHP // 2, _row, 0)

    def _out_issue(jj, a_v, sem):
        pltpu.async_copy(a_v, out.at[pl.ds((wid * CPTP + jj) * CHP, CHP)], sem)

    def _out_wait(jj, a_v, sem):
        pltpu.make_async_copy(a_v, out.at[pl.ds((wid * CPTP + jj) * CHP, CHP)],
                              sem).wait()

    _issue(0, a_a, b_a, sem_a)
    _issue(1, a_b, b_b, sem_b)

    def _pair(t, carry):
        ja = 2 * t
        jb = 2 * t + 1
        _wait(ja, a_a, b_a, sem_a)
        _add(a_a, b_a)
        _out_issue(ja, a_a, sem_oa)
        _wait(jb, a_b, b_b, sem_b)
        _add(a_b, b_b)
        _out_issue(jb, a_b, sem_ob)
        _out_wait(ja, a_a, sem_oa)

        @pl.when(jb + 1 < CPTP)
        def _():
            _issue(jb + 1, a_a, b_a, sem_a)

        _out_wait(jb, a_b, sem_ob)

        @pl.when(jb + 2 < CPTP)
        def _():
            _issue(jb + 2, a_b, b_b, sem_b)
        return carry
    lax.fori_loop(0, CPTP // 2, _pair, 0)


_pred_sc = pl.kernel(
    _pred_sc_body,
    out_type=jax.ShapeDtypeStruct((E_PAD, 16), _f32),
    mesh=_MESH,
    compiler_params=pltpu.CompilerParams(use_tc_tiling_on_sc=False),
    scratch_types=[
        pltpu.VMEM((CPTP, CHP), jnp.int32),
        pltpu.VMEM((CPTP, CHP), jnp.int32),
        pltpu.VMEM((CHP, 16), _f32),
        pltpu.VMEM((CHP, 16), _f32),
        pltpu.VMEM((CHP, 16), _f32),
        pltpu.VMEM((CHP, 16), _f32),
        pltpu.SemaphoreType.DMA,
        pltpu.SemaphoreType.DMA,
        pltpu.SemaphoreType.DMA,
        pltpu.SemaphoreType.DMA,
    ],
)


def kernel(nfeats, efeats, edge_index, Wm1, bm1, Wa1, ba1, Wm2, bm2, Wa2,
           ba2, Wp, bp):
    nf = nfeats[:, 0, :]
    ef = efeats[:, 0, :]
    src = edge_index[0]
    dst = edge_index[1]

    pad = E_PAD - E
    src_p = jnp.concatenate([src, jnp.zeros((pad,), jnp.int32)])
    dst_p = jnp.concatenate([dst, jnp.full((pad,), N, jnp.int32)])
    srcs2 = src_p.reshape(NCHUNKS, CH)
    dsts2 = dst_p.reshape(NCHUNKS, CH)
    srcsP = src_p.reshape(E_PAD // CHP, CHP)
    dstsP = dst_p.reshape(E_PAD // CHP, CHP)
    nf_p = jnp.concatenate([nf, jnp.zeros((N_ACC - N, DIN), _f32)], axis=0)

    Wm1xT = Wm1[:, :DIN].T
    Wm1eT = Wm1[:, DIN:].T
    Wm2xT = Wm2[:, :DHID].T
    Wm2eT = Wm2[:, DHID:].T
    Wa1xT = Wa1[:, :DIN].T
    Wa1hT = Wa1[:, DIN:].T
    Wa2xT = Wa2[:, :DHID].T
    Wa2hT = Wa2[:, DHID:].T
    WpuT = jnp.zeros((DOUT, 16), _f32).at[:, :NCLS].set(Wp[:, :DOUT].T)
    WpvT = jnp.zeros((DOUT, 16), _f32).at[:, :NCLS].set(Wp[:, DOUT:].T)
    bp16 = jnp.zeros((1, 16), _f32).at[0, :NCLS].set(bp)
    bm1r = bm1[None, :]
    bm2r = bm2[None, :]
    ba1r = ba1[None, :]
    ba2r = ba2[None, :]

    z128 = jnp.zeros((N_ACC, 128), _f32)
    z16 = jnp.zeros((N_ACC, 16), _f32)

    nfw1 = pl.pallas_call(
        _nfw_tc,
        grid=(N_ACC // _NBLK,),
        in_specs=[_row_spec(_NBLK, 128), _full_spec(128, 128)],
        out_specs=_row_spec(_NBLK, 128),
        out_shape=jax.ShapeDtypeStruct((N_ACC, 128), _f32),
    )(nf_p, Wm1xT)

    efw1, efw2 = pl.pallas_call(
        _efw_tc,
        grid=(E // _EBLK,),
        in_specs=[_row_spec(_EBLK, EDIM), _full_spec(EDIM, 128),
                  _full_spec(1, 128), _full_spec(EDIM, 128),
                  _full_spec(1, 128)],
        out_specs=[_row_spec(_EBLK, 128), _row_spec(_EBLK, 128)],
        out_shape=[jax.ShapeDtypeStruct((E_PAD, 128), _f32),
                   jax.ShapeDtypeStruct((E_PAD, 128), _f32)],
    )(ef, Wm1eT, bm1r, Wm2eT, bm2r)

    c1p = _cnt_sc(dsts2, z16)

    s1p = _sage_sc(nfw1, efw1, srcs2, dsts2, z128)

    h1, nfw2, inv16 = pl.pallas_call(
        _mid_tc,
        grid=(N_ACC // _NBLK,),
        in_specs=[_row_spec(_NBLK, 128), _pair_spec(_NBLK, 128),
                  _pair_spec(_NBLK, 16), _full_spec(128, 128),
                  _full_spec(128, 128), _full_spec(1, 128),
                  _full_spec(128, 128)],
        out_specs=[_row_spec(_NBLK, 128), _row_spec(_NBLK, 128),
                   _row_spec(_NBLK, 16)],
        out_shape=[jax.ShapeDtypeStruct((N_ACC, 128), _f32),
                   jax.ShapeDtypeStruct((N_ACC, 128), _f32),
                   jax.ShapeDtypeStruct((N_ACC, 16), _f32)],
    )(nf_p, s1p, c1p, Wa1xT, Wa1hT, ba1r, Wm2xT)

    s2p = _sage_sc(nfw2, efw2, srcs2, dsts2, z128)

    pu, pv = pl.pallas_call(
        _fin_tc,
        grid=(N_ACC // _NBLK,),
        in_specs=[_row_spec(_NBLK, 128), _pair_spec(_NBLK, 128),
                  _row_spec(_NBLK, 16),
                  _full_spec(128, 128), _full_spec(128, 128),
                  _full_spec(1, 128), _full_spec(128, 16),
                  _full_spec(128, 16), _full_spec(1, 16)],
        out_specs=[_row_spec(_NBLK, 16), _row_spec(_NBLK, 16)],
        out_shape=[jax.ShapeDtypeStruct((N_ACC, 16), _f32),
                   jax.ShapeDtypeStruct((N_ACC, 16), _f32)],
    )(h1, s2p, inv16, Wa2xT, Wa2hT, ba2r, WpuT, WpvT, bp16)

    score16 = _pred_sc(pu, pv, srcsP, dstsP)
    return score16[:E, :NCLS]


# speedup vs baseline: 1.0169x; 1.0169x over previous
import jax
import jax.numpy as jnp
from jax import lax
from jax.experimental import pallas as pl
from jax.experimental.pallas import tpu as pltpu
from jax.experimental.pallas import tpu_sc as plsc

N = 10000
E = 320000
DIN = 128
DHID = 128
DOUT = 128
EDIM = 16
NCLS = 10

NC = 2
NS = 16
NW = NC * NS
CH = 64
CPT = 160
GRP = 32
E_PAD = NW * CPT * CH
NCHUNKS = NW * CPT
N_ACC = 10112
RPT = N_ACC // NS

_f32 = jnp.float32


def _nfw_tc(x_ref, w_ref, o_ref):
    o_ref[...] = jnp.dot(x_ref[...], w_ref[...], preferred_element_type=_f32)


def _efw_tc(ef_ref, w1_ref, b1_ref, w2_ref, b2_ref, o1_ref, o2_ref):
    ef = ef_ref[...]
    o1_ref[...] = jnp.dot(ef, w1_ref[...], preferred_element_type=_f32) + b1_ref[...]
    o2_ref[...] = jnp.dot(ef, w2_ref[...], preferred_element_type=_f32) + b2_ref[...]


def _mid_tc(nf_ref, sp_ref, cp_ref, waxT_ref, wahT_ref,
            ba_ref, wm2xT_ref, h1_o, nfw2_o, inv_o):
    cnt = cp_ref[0][:, 0:1] + cp_ref[1][:, 0:1]
    inv = 1.0 / jnp.maximum(cnt, 1.0)
    hn = (sp_ref[0] + sp_ref[1]) * inv
    h1 = jnp.maximum(
        jnp.dot(nf_ref[...], waxT_ref[...], preferred_element_type=_f32)
        + jnp.dot(hn, wahT_ref[...], preferred_element_type=_f32)
        + ba_ref[...], 0.0)
    h1_o[...] = h1
    nfw2_o[...] = jnp.dot(h1, wm2xT_ref[...], preferred_element_type=_f32)
    inv_o[...] = jnp.broadcast_to(inv, inv_o.shape)


def _fin_tc(h1_ref, sp_ref, inv_ref, waxT_ref, wahT_ref, ba_ref,
            wpuT_ref, wpvT_ref, bp_ref, pu_o, pv_o):
    hn = (sp_ref[0] + sp_ref[1]) * inv_ref[...][:, 0:1]
    h2 = jnp.maximum(
        jnp.dot(h1_ref[...], waxT_ref[...], preferred_element_type=_f32)
        + jnp.dot(hn, wahT_ref[...], preferred_element_type=_f32)
        + ba_ref[...], 0.0)
    pu_o[...] = jnp.dot(h2, wpuT_ref[...], preferred_element_type=_f32) + bp_ref[...]
    pv_o[...] = jnp.dot(h2, wpvT_ref[...], preferred_element_type=_f32)


_NBLK = 1264
_EBLK = 2000


def _row_spec(blk, cols):
    return pl.BlockSpec((blk, cols), lambda i: (i, 0))


def _pair_spec(blk, cols):
    return pl.BlockSpec((2, blk, cols), lambda i: (0, i, 0))


def _full_spec(r, c):
    return pl.BlockSpec((r, c), lambda i: (0, 0))


_MESH = plsc.VectorSubcoreMesh(core_axis_name="c", subcore_axis_name="s",
                               num_cores=NC, num_subcores=NS)


def _sage_sc_body(nfw, efw, srcs, dsts, z128,
                  out,
                  src_v, dst_v, g_a, e_a, g_b, e_b, acc,
                  sem_a, sem_b):
    c = lax.axis_index("c")
    s = lax.axis_index("s")
    wid = s * NC + c
    r0 = s * RPT

    pltpu.sync_copy(z128.at[pl.ds(r0, RPT)], acc.at[pl.ds(r0, RPT)])
    plsc.subcore_barrier()

    def _issue(jj, cbase, g_v, e_v, sem):
        base = (cbase + jj) * CH
        pltpu.async_copy(efw.at[pl.ds(base, CH)], e_v, sem)
        pltpu.async_copy(nfw.at[src_v.at[jj]], g_v, sem)

    def _wait(jj, cbase, g_v, e_v, sem):
        base = (cbase + jj) * CH
        pltpu.make_async_copy(efw.at[pl.ds(base, CH)], e_v, sem).wait()
        pltpu.make_async_copy(nfw.at[src_v.at[jj]], g_v, sem).wait()

    def _compute(g_v, e_v):
        def _row(i, carry2):
            for r in range(2):
                for k in range(8):
                    sl = pl.ds(k * 16, 16)
                    g_v[2 * i + r, sl] = jnp.maximum(
                        g_v[2 * i + r, sl] + e_v[2 * i + r, sl], 0.0)
            return carry2
        lax.fori_loop(0, CH // 2, _row, 0)

    def _group(g, carry):
        cbase = wid * CPT + g * GRP
        pltpu.sync_copy(srcs.at[pl.ds(cbase, GRP)], src_v)
        pltpu.sync_copy(dsts.at[pl.ds(cbase, GRP)], dst_v)
        _issue(0, cbase, g_a, e_a, sem_a)

        def _pair(t, carry1):
            ja = 2 * t
            jb = 2 * t + 1
            _issue(jb, cbase, g_b, e_b, sem_b)
            _wait(ja, cbase, g_a, e_a, sem_a)
            _compute(g_a, e_a)
            pltpu.sync_copy(g_a, acc.at[dst_v.at[ja]], add=True)

            @pl.when(jb + 1 < GRP)
            def _():
                _issue(jb + 1, cbase, g_a, e_a, sem_a)

            _wait(jb, cbase, g_b, e_b, sem_b)
            _compute(g_b, e_b)
            pltpu.sync_copy(g_b, acc.at[dst_v.at[jb]], add=True)
            return carry1
        lax.fori_loop(0, GRP // 2, _pair, 0)
        return carry
    lax.fori_loop(0, CPT // GRP, _group, 0)

    plsc.subcore_barrier()
    pltpu.sync_copy(acc.at[pl.ds(r0, RPT)], out.at[c, pl.ds(r0, RPT)])


_sage_sc = pl.kernel(
    _sage_sc_body,
    out_type=jax.ShapeDtypeStruct((NC, N_ACC, 128), _f32),
    mesh=_MESH,
    compiler_params=pltpu.CompilerParams(use_tc_tiling_on_sc=False),
    scratch_types=[
        pltpu.VMEM((GRP, CH), jnp.int32),
        pltpu.VMEM((GRP, CH), jnp.int32),
        pltpu.VMEM((CH, 128), _f32),
        pltpu.VMEM((CH, 128), _f32),
        pltpu.VMEM((CH, 128), _f32),
        pltpu.VMEM((CH, 128), _f32),
        pltpu.VMEM_SHARED((N_ACC, 128), _f32),
        pltpu.SemaphoreType.DMA,
        pltpu.SemaphoreType.DMA,
    ],
)


def _cnt_sc_body(dsts, z16, cnt_out, dst_v, ones_v, cacc):
    c = lax.axis_index("c")
    s = lax.axis_index("s")
    wid = s * NC + c
    r0 = s * RPT

    pltpu.sync_copy(z16.at[pl.ds(r0, RPT)], cacc.at[pl.ds(r0, RPT)])

    def _ones_row(i, carry):
        ones_v[i, :] = jnp.ones((16,), _f32)
        return carry
    lax.fori_loop(0, CH, _ones_row, 0)

    pltpu.sync_copy(dsts.at[pl.ds(wid * CPT, CPT)], dst_v)
    plsc.subcore_barrier()

    def _chunk(j, carry):
        pltpu.sync_copy(ones_v, cacc.at[dst_v.at[j]], add=True)
        return carry
    lax.fori_loop(0, CPT, _chunk, 0)

    plsc.subcore_barrier()
    pltpu.sync_copy(cacc.at[pl.ds(r0, RPT)], cnt_out.at[c, pl.ds(r0, RPT)])


_cnt_sc = pl.kernel(
    _cnt_sc_body,
    out_type=jax.ShapeDtypeStruct((NC, N_ACC, 16), _f32),
    mesh=_MESH,
    compiler_params=pltpu.CompilerParams(use_tc_tiling_on_sc=False),
    scratch_types=[
        pltpu.VMEM((CPT, CH), jnp.int32),
        pltpu.VMEM((CH, 16), _f32),
        pltpu.VMEM_SHARED((N_ACC, 16), _f32),
    ],
)


CHP = 128
CPTP = E_PAD // (NW * CHP)


def _pred_sc_body(pu, pv, srcs, dsts, out,
                  src_v, dst_v, a_a, b_a, a_b, b_b,
                  sem_a, sem_b, sem_oa, sem_ob):
    c = lax.axis_index("c")
    s = lax.axis_index("s")
    wid = s * NC + c

    pltpu.sync_copy(srcs.at[pl.ds(wid * CPTP, CPTP)], src_v)
    pltpu.sync_copy(dsts.at[pl.ds(wid * CPTP, CPTP)], dst_v)

    def _issue(jj, a_v, b_v, sem):
        pltpu.async_copy(pu.at[src_v.at[jj]], a_v, sem)
        pltpu.async_copy(pv.at[dst_v.at[jj]], b_v, sem)

    def _wait(jj, a_v, b_v, sem):
        pltpu.make_async_copy(pu.at[src_v.at[jj]], a_v, sem).wait()
        pltpu.make_async_copy(pv.at[dst_v.at[jj]], b_v, sem).wait()

    def _add(a_v, b_v):
        def _row(i, carry2):
            for r in range(2):
                a_v[2 * i + r, :] = a_v[2 * i + r, :] + b_v[2 * i + r, :]
            return carry2
        lax.fori_loop(0, CHP // 2, _row, 0)

    def _out_issue(jj, a_v, sem):
        pltpu.async_copy(a_v, out.at[pl.ds((wid * CPTP + jj) * CHP, CHP)], sem)

    def _out_wait(jj, a_v, sem):
        pltpu.make_async_copy(a_v, out.at[pl.ds((wid * CPTP + jj) * CHP, CHP)],
                              sem).wait()

    _issue(0, a_a, b_a, sem_a)
    _issue(1, a_b, b_b, sem_b)

    def _pair(t, carry):
        ja = 2 * t
        jb = 2 * t + 1
        _wait(ja, a_a, b_a, sem_a)
        _add(a_a, b_a)
        _out_issue(ja, a_a, sem_oa)
        _wait(jb, a_b, b_b, sem_b)
        _add(a_b, b_b)
        _out_issue(jb, a_b, sem_ob)
        _out_wait(ja, a_a, sem_oa)

        @pl.when(jb + 1 < CPTP)
        def _():
            _issue(jb + 1, a_a, b_a, sem_a)

        _out_wait(jb, a_b, sem_ob)

        @pl.when(jb + 2 < CPTP)
        def _():
            _issue(jb + 2, a_b, b_b, sem_b)
        return carry
    lax.fori_loop(0, CPTP // 2, _pair, 0)


_pred_sc = pl.kernel(
    _pred_sc_body,
    out_type=jax.ShapeDtypeStruct((E_PAD, 16), _f32),
    mesh=_MESH,
    compiler_params=pltpu.CompilerParams(use_tc_tiling_on_sc=False),
    scratch_types=[
        pltpu.VMEM((CPTP, CHP), jnp.int32),
        pltpu.VMEM((CPTP, CHP), jnp.int32),
        pltpu.VMEM((CHP, 16), _f32),
        pltpu.VMEM((CHP, 16), _f32),
        pltpu.VMEM((CHP, 16), _f32),
        pltpu.VMEM((CHP, 16), _f32),
        pltpu.SemaphoreType.DMA,
        pltpu.SemaphoreType.DMA,
        pltpu.SemaphoreType.DMA,
        pltpu.SemaphoreType.DMA,
    ],
)


def kernel(nfeats, efeats, edge_index, Wm1, bm1, Wa1, ba1, Wm2, bm2, Wa2,
           ba2, Wp, bp):
    nf = nfeats[:, 0, :]
    ef = efeats[:, 0, :]
    src = edge_index[0]
    dst = edge_index[1]

    pad = E_PAD - E
    src_p = jnp.concatenate([src, jnp.zeros((pad,), jnp.int32)])
    dst_p = jnp.concatenate([dst, jnp.full((pad,), N, jnp.int32)])
    srcs2 = src_p.reshape(NCHUNKS, CH)
    dsts2 = dst_p.reshape(NCHUNKS, CH)
    srcsP = src_p.reshape(E_PAD // CHP, CHP)
    dstsP = dst_p.reshape(E_PAD // CHP, CHP)
    nf_p = jnp.concatenate([nf, jnp.zeros((N_ACC - N, DIN), _f32)], axis=0)

    Wm1xT = Wm1[:, :DIN].T
    Wm1eT = Wm1[:, DIN:].T
    Wm2xT = Wm2[:, :DHID].T
    Wm2eT = Wm2[:, DHID:].T
    Wa1xT = Wa1[:, :DIN].T
    Wa1hT = Wa1[:, DIN:].T
    Wa2xT = Wa2[:, :DHID].T
    Wa2hT = Wa2[:, DHID:].T
    WpuT = jnp.zeros((DOUT, 16), _f32).at[:, :NCLS].set(Wp[:, :DOUT].T)
    WpvT = jnp.zeros((DOUT, 16), _f32).at[:, :NCLS].set(Wp[:, DOUT:].T)
    bp16 = jnp.zeros((1, 16), _f32).at[0, :NCLS].set(bp)
    bm1r = bm1[None, :]
    bm2r = bm2[None, :]
    ba1r = ba1[None, :]
    ba2r = ba2[None, :]

    z128 = jnp.zeros((N_ACC, 128), _f32)
    z16 = jnp.zeros((N_ACC, 16), _f32)

    nfw1 = pl.pallas_call(
        _nfw_tc,
        grid=(N_ACC // _NBLK,),
        in_specs=[_row_spec(_NBLK, 128), _full_spec(128, 128)],
        out_specs=_row_spec(_NBLK, 128),
        out_shape=jax.ShapeDtypeStruct((N_ACC, 128), _f32),
    )(nf_p, Wm1xT)

    efw1, efw2 = pl.pallas_call(
        _efw_tc,
        grid=(E // _EBLK,),
        in_specs=[_row_spec(_EBLK, EDIM), _full_spec(EDIM, 128),
                  _full_spec(1, 128), _full_spec(EDIM, 128),
                  _full_spec(1, 128)],
        out_specs=[_row_spec(_EBLK, 128), _row_spec(_EBLK, 128)],
        out_shape=[jax.ShapeDtypeStruct((E_PAD, 128), _f32),
                   jax.ShapeDtypeStruct((E_PAD, 128), _f32)],
    )(ef, Wm1eT, bm1r, Wm2eT, bm2r)

    c1p = _cnt_sc(dsts2, z16)

    s1p = _sage_sc(nfw1, efw1, srcs2, dsts2, z128)

    h1, nfw2, inv16 = pl.pallas_call(
        _mid_tc,
        grid=(N_ACC // _NBLK,),
        in_specs=[_row_spec(_NBLK, 128), _pair_spec(_NBLK, 128),
                  _pair_spec(_NBLK, 16), _full_spec(128, 128),
                  _full_spec(128, 128), _full_spec(1, 128),
                  _full_spec(128, 128)],
        out_specs=[_row_spec(_NBLK, 128), _row_spec(_NBLK, 128),
                   _row_spec(_NBLK, 16)],
        out_shape=[jax.ShapeDtypeStruct((N_ACC, 128), _f32),
                   jax.ShapeDtypeStruct((N_ACC, 128), _f32),
                   jax.ShapeDtypeStruct((N_ACC, 16), _f32)],
    )(nf_p, s1p, c1p, Wa1xT, Wa1hT, ba1r, Wm2xT)

    s2p = _sage_sc(nfw2, efw2, srcs2, dsts2, z128)

    pu, pv = pl.pallas_call(
        _fin_tc,
        grid=(N_ACC // _NBLK,),
        in_specs=[_row_spec(_NBLK, 128), _pair_spec(_NBLK, 128),
                  _row_spec(_NBLK, 16),
                  _full_spec(128, 128), _full_spec(128, 128),
                  _full_spec(1, 128), _full_spec(128, 16),
                  _full_spec(128, 16), _full_spec(1, 16)],
        out_specs=[_row_spec(_NBLK, 16), _row_spec(_NBLK, 16)],
        out_shape=[jax.ShapeDtypeStruct((N_ACC, 16), _f32),
                   jax.ShapeDtypeStruct((N_ACC, 16), _f32)],
    )(h1, s2p, inv16, Wa2xT, Wa2hT, ba2r, WpuT, WpvT, bp16)

    score16 = _pred_sc(pu, pv, srcsP, dstsP)
    return score16[:E, :NCLS]
